# baseline (speedup 1.0000x reference)
import functools

import jax
import jax.numpy as jnp
from jax import lax
from jax.experimental import pallas as pl
from jax.experimental.pallas import tpu as pltpu

_DIV_EPS = 1e-6
_BN_EPS = 1e-5


def _recip(x):
    r = pl.reciprocal(x, approx=True)
    return r * (2.0 - x * r)


def _pass1_kernel(nodebf_ref, node_ref, edge_ref, src_ref, dstr_ref,
                  wa_ref, wdg_ref, weg_ref, bm_ref, bdu_ref,
                  wsu_ref, bsu_ref,
                  gn_ref, bn_ref, ge_ref, be_ref,
                  m_out, x_out, ss_out,
                  acc_ref, st_ref,
                  *, num_edges, edge_tile, padded):
    f32 = jnp.float32
    bf16 = jnp.bfloat16
    N = nodebf_ref.shape[0]
    F = wdg_ref.shape[0]
    TE = edge_tile
    t = pl.program_id(0)
    last = pl.num_programs(0) - 1

    @pl.when(t == 0)
    def _init():
        acc_ref[...] = jnp.zeros_like(acc_ref)
        st_ref[...] = jnp.zeros_like(st_ref)

    TE2 = TE // 2
    subl = lax.broadcasted_iota(jnp.int32, (N, TE2), 0)
    dnt = (((0,), (0,)), ((), ()))
    scat = []
    for h in range(2):
        rows = pl.ds(h * TE2, TE2)
        St = (subl == src_ref[0, h:h + 1, :]).astype(bf16)
        Dt = (subl == dstr_ref[0, h:h + 1, :]).astype(bf16)

        gS = lax.dot_general(St, nodebf_ref[...], dnt,
                             preferred_element_type=f32)
        gD = lax.dot_general(Dt, nodebf_ref[...], dnt,
                             preferred_element_type=f32)

        P1 = jnp.dot(gS.astype(bf16), wa_ref[...],
                     preferred_element_type=f32)
        P2 = jnp.dot(gD.astype(bf16), wdg_ref[...],
                     preferred_element_type=f32)
        P3 = jnp.dot(edge_ref[rows, :].astype(bf16), weg_ref[...],
                     preferred_element_type=f32)
        m = P1[:, 0:F] + P2 + P3 + bm_ref[...]
        m_out[rows, :] = m.astype(bf16)

        if padded:
            eidx = (t * TE + h * TE2
                    + lax.broadcasted_iota(jnp.int32, (TE2, 1), 0))
            valid = (eidx < num_edges).astype(f32)
            mv = m * valid
        else:
            valid = None
            mv = m

        st_ref[0:1, :] += jnp.sum(mv, axis=0, keepdims=True)
        st_ref[1:2, :] += jnp.sum(mv * m, axis=0, keepdims=True)

        sigma = 0.5 * jnp.tanh(0.5 * m) + 0.5
        if padded:
            sigma = sigma * valid

        Bh = P1[:, F:2 * F] + bdu_ref[...]
        contrib = jnp.concatenate([Bh * sigma, sigma], axis=1).astype(bf16)
        scat.append(jnp.dot(Dt, contrib, preferred_element_type=f32))

    acc_ref[...] += scat[0] + scat[1]

    @pl.when(t == last)
    def _finalize():
        xl = jnp.dot(nodebf_ref[...], wsu_ref[...],
                     preferred_element_type=f32) + bsu_ref[...]
        num = acc_ref[:, 0:F]
        den = acc_ref[:, F:2 * F]
        x = xl + num * _recip(den + _DIV_EPS)
        mean = jnp.mean(x, axis=0, keepdims=True)
        var = jnp.mean((x - mean) ** 2, axis=0, keepdims=True)
        xn = (x - mean) * lax.rsqrt(var + _BN_EPS) * gn_ref[...] + bn_ref[...]
        x_out[...] = node_ref[...] + jnp.clip(xn, 0.0, 6.0)

        inv_e = 1.0 / float(num_edges)
        s = st_ref[0:1, :] * inv_e
        q = st_ref[1:2, :] * inv_e
        var_e = q - s * s
        scale = ge_ref[...] * lax.rsqrt(var_e + _BN_EPS)
        ss_out[0:1, :] = scale
        ss_out[1:2, :] = be_ref[...] - s * scale
        ss_out[2:8, :] = jnp.zeros((6, F), f32)


def _edge_out_kernel(edge_ref, m_ref, ss_ref, y_out):
    y_out[...] = edge_ref[...] + jnp.clip(
        m_ref[...].astype(jnp.float32) * ss_ref[0:1, :] + ss_ref[1:2, :],
        0.0, 6.0)


def _pick_tile(n, cap, step=8):
    best = 0
    d = step
    while d <= min(n, cap):
        if n % d == 0:
            best = d
        d += step
    return best


def kernel(node_feats, edge_feats, src, dst,
           w_src_gate, b_src_gate, w_dst_gate, b_dst_gate,
           w_edge_gate, b_edge_gate, w_src_update, b_src_update,
           w_dst_update, b_dst_update,
           bn_nodes_gamma, bn_nodes_beta, bn_edges_gamma, bn_edges_beta):
    f32 = jnp.float32
    bf16 = jnp.bfloat16
    N, F = node_feats.shape
    E = edge_feats.shape[0]

    TE = _pick_tile(E, cap=2048)
    if TE >= 256:
        E_pad, padded = E, False
        edge_p, src_p, dst_p = edge_feats, src, dst
    else:
        TE = 1024
        E_pad = ((E + TE - 1) // TE) * TE
        padded = True
        edge_p = jnp.pad(edge_feats, ((0, E_pad - E), (0, 0)))
        src_p = jnp.pad(src, (0, E_pad - E))
        dst_p = jnp.pad(dst, (0, E_pad - E))
    n_t = E_pad // TE
    src_r = src_p.astype(jnp.int32).reshape(n_t, 2, TE // 2)
    dst_r = dst_p.astype(jnp.int32).reshape(n_t, 2, TE // 2)
    n_tiles = E_pad // TE

    node_bf = node_feats.astype(bf16)
    wa = jnp.concatenate([w_src_gate, w_dst_update], axis=1).astype(bf16)
    bm = b_src_gate + b_dst_gate + b_edge_gate

    m, x, ss = pl.pallas_call(
        functools.partial(_pass1_kernel, num_edges=E, edge_tile=TE,
                          padded=padded),
        out_shape=(jax.ShapeDtypeStruct((E_pad, F), bf16),
                   jax.ShapeDtypeStruct((N, F), f32),
                   jax.ShapeDtypeStruct((8, F), f32)),
        grid=(n_tiles,),
        in_specs=[
            pl.BlockSpec((N, F), lambda t: (0, 0)),
            pl.BlockSpec((N, F), lambda t: (0, 0)),
            pl.BlockSpec((TE, F), lambda t: (t, 0)),
            pl.BlockSpec((1, 2, TE // 2), lambda t: (t, 0, 0)),
            pl.BlockSpec((1, 2, TE // 2), lambda t: (t, 0, 0)),
            pl.BlockSpec((F, 2 * F), lambda t: (0, 0)),
            pl.BlockSpec((F, F), lambda t: (0, 0)),
            pl.BlockSpec((F, F), lambda t: (0, 0)),
            pl.BlockSpec((1, F), lambda t: (0, 0)),
            pl.BlockSpec((1, F), lambda t: (0, 0)),
            pl.BlockSpec((F, F), lambda t: (0, 0)),
            pl.BlockSpec((1, F), lambda t: (0, 0)),
            pl.BlockSpec((1, F), lambda t: (0, 0)),
            pl.BlockSpec((1, F), lambda t: (0, 0)),
            pl.BlockSpec((1, F), lambda t: (0, 0)),
            pl.BlockSpec((1, F), lambda t: (0, 0)),
        ],
        out_specs=(
            pl.BlockSpec((TE, F), lambda t: (t, 0)),
            pl.BlockSpec((N, F), lambda t: (0, 0)),
            pl.BlockSpec((8, F), lambda t: (0, 0)),
        ),
        scratch_shapes=[pltpu.VMEM((N, 2 * F), f32),
                        pltpu.VMEM((8, F), f32)],
        compiler_params=pltpu.CompilerParams(
            dimension_semantics=("arbitrary",),
            vmem_limit_bytes=56 * 1024 * 1024),
    )(node_bf, node_feats, edge_p, src_r, dst_r,
      wa, w_dst_gate.astype(bf16), w_edge_gate.astype(bf16), bm, b_dst_update,
      w_src_update.astype(bf16), b_src_update,
      bn_nodes_gamma, bn_nodes_beta, bn_edges_gamma, bn_edges_beta)

    E4 = E_pad // 4
    edge4 = edge_p.reshape(E4, 4 * F)
    m4 = m.reshape(E4, 4 * F)
    ss4 = jnp.tile(ss, (1, 4))
    tb = _pick_tile(E4, cap=1024)
    if tb == 0:
        tb = E4
    y4 = pl.pallas_call(
        _edge_out_kernel,
        out_shape=jax.ShapeDtypeStruct((E4, 4 * F), f32),
        grid=(E4 // tb,),
        in_specs=[
            pl.BlockSpec((tb, 4 * F), lambda t: (t, 0)),
            pl.BlockSpec((tb, 4 * F), lambda t: (t, 0)),
            pl.BlockSpec((8, 4 * F), lambda t: (0, 0)),
        ],
        out_specs=pl.BlockSpec((tb, 4 * F), lambda t: (t, 0)),
        compiler_params=pltpu.CompilerParams(
            dimension_semantics=("arbitrary",)),
    )(edge4, m4, ss4)
    y = y4.reshape(E_pad, F)[:E]
    return x, y


# speedup vs baseline: 1.4753x; 1.4753x over previous
import functools

import jax
import jax.numpy as jnp
from jax import lax
from jax.experimental import pallas as pl
from jax.experimental.pallas import tpu as pltpu

_DIV_EPS = 1e-6
_BN_EPS = 1e-5


def _recip(x):
    r = pl.reciprocal(x, approx=True)
    return r * (2.0 - x * r)


def _pass1_kernel(nodebf_ref, node_ref, edge_ref, src_ref, dstr_ref,
                  wa_ref, wdg_ref, weg_ref, bm_ref, bdu_ref,
                  wsu_ref, bsu_ref,
                  gn_ref, bn_ref, ge_ref, be_ref,
                  m_out, x_out, ss_out,
                  acc_ref, st_ref,
                  *, num_edges, edge_tile, padded):
    f32 = jnp.float32
    bf16 = jnp.bfloat16
    N = nodebf_ref.shape[0]
    F = wdg_ref.shape[0]
    TE = edge_tile
    t = pl.program_id(0)
    last = pl.num_programs(0) - 1

    @pl.when(t == 0)
    def _init():
        acc_ref[...] = jnp.zeros_like(acc_ref)
        st_ref[...] = jnp.zeros_like(st_ref)

    TE2 = TE // 2
    subl = lax.broadcasted_iota(jnp.int32, (N, TE2), 0)
    dnt = (((0,), (0,)), ((), ()))
    scat = []
    for h in range(2):
        rows = pl.ds(h * TE2, TE2)
        St = (subl == src_ref[0, h:h + 1, :]).astype(bf16)
        Dt = (subl == dstr_ref[0, h:h + 1, :]).astype(bf16)

        gS = lax.dot_general(St, nodebf_ref[...], dnt,
                             preferred_element_type=f32)
        gD = lax.dot_general(Dt, nodebf_ref[...], dnt,
                             preferred_element_type=f32)

        P1 = jnp.dot(gS.astype(bf16), wa_ref[...],
                     preferred_element_type=f32)
        P2 = jnp.dot(gD.astype(bf16), wdg_ref[...],
                     preferred_element_type=f32)
        P3 = jnp.dot(edge_ref[rows, :].astype(bf16), weg_ref[...],
                     preferred_element_type=f32)
        m = P1[:, 0:F] + P2 + P3 + bm_ref[...]
        m_out[rows, :] = m.astype(bf16)

        if padded:
            eidx = (t * TE + h * TE2
                    + lax.broadcasted_iota(jnp.int32, (TE2, 1), 0))
            valid = (eidx < num_edges).astype(f32)
            mv = m * valid
        else:
            valid = None
            mv = m

        st_ref[0:1, :] += jnp.sum(mv, axis=0, keepdims=True)
        st_ref[1:2, :] += jnp.sum(mv * m, axis=0, keepdims=True)

        sigma = 0.5 * jnp.tanh(0.5 * m) + 0.5
        if padded:
            sigma = sigma * valid

        Bh = P1[:, F:2 * F] + bdu_ref[...]
        contrib = jnp.concatenate([Bh * sigma, sigma], axis=1).astype(bf16)
        scat.append(jnp.dot(Dt, contrib, preferred_element_type=f32))

    acc_ref[...] += scat[0] + scat[1]

    @pl.when(t == last)
    def _finalize():
        xl = jnp.dot(nodebf_ref[...], wsu_ref[...],
                     preferred_element_type=f32) + bsu_ref[...]
        num = acc_ref[:, 0:F]
        den = acc_ref[:, F:2 * F]
        x = xl + num * _recip(den + _DIV_EPS)
        mean = jnp.mean(x, axis=0, keepdims=True)
        var = jnp.mean((x - mean) ** 2, axis=0, keepdims=True)
        xn = (x - mean) * lax.rsqrt(var + _BN_EPS) * gn_ref[...] + bn_ref[...]
        x_out[...] = node_ref[...] + jnp.clip(xn, 0.0, 6.0)

        inv_e = 1.0 / float(num_edges)
        s = st_ref[0:1, :] * inv_e
        q = st_ref[1:2, :] * inv_e
        var_e = q - s * s
        scale = ge_ref[...] * lax.rsqrt(var_e + _BN_EPS)
        ss_out[0:1, :] = scale
        ss_out[1:2, :] = be_ref[...] - s * scale
        ss_out[2:8, :] = jnp.zeros((6, F), f32)


def _edge_out_kernel(edge_ref, m_ref, ss_ref, y_out):
    y_out[...] = edge_ref[...] + jnp.clip(
        m_ref[...].astype(jnp.float32) * ss_ref[0:1, :] + ss_ref[1:2, :],
        0.0, 6.0)


def _pick_tile(n, cap, step=8):
    best = 0
    d = step
    while d <= min(n, cap):
        if n % d == 0:
            best = d
        d += step
    return best


def kernel(node_feats, edge_feats, src, dst,
           w_src_gate, b_src_gate, w_dst_gate, b_dst_gate,
           w_edge_gate, b_edge_gate, w_src_update, b_src_update,
           w_dst_update, b_dst_update,
           bn_nodes_gamma, bn_nodes_beta, bn_edges_gamma, bn_edges_beta):
    f32 = jnp.float32
    bf16 = jnp.bfloat16
    N, F = node_feats.shape
    E = edge_feats.shape[0]

    TE = _pick_tile(E, cap=2048)
    if TE >= 256:
        E_pad, padded = E, False
        edge_p, src_p, dst_p = edge_feats, src, dst
    else:
        TE = 1024
        E_pad = ((E + TE - 1) // TE) * TE
        padded = True
        edge_p = jnp.pad(edge_feats, ((0, E_pad - E), (0, 0)))
        src_p = jnp.pad(src, (0, E_pad - E))
        dst_p = jnp.pad(dst, (0, E_pad - E))
    n_t = E_pad // TE
    src_r = src_p.astype(jnp.int32).reshape(n_t, 2, TE // 2)
    dst_r = dst_p.astype(jnp.int32).reshape(n_t, 2, TE // 2)
    n_tiles = E_pad // TE

    node_bf = node_feats.astype(bf16)
    wa = jnp.concatenate([w_src_gate, w_dst_update], axis=1).astype(bf16)
    bm = b_src_gate + b_dst_gate + b_edge_gate

    m, x, ss = pl.pallas_call(
        functools.partial(_pass1_kernel, num_edges=E, edge_tile=TE,
                          padded=padded),
        out_shape=(jax.ShapeDtypeStruct((E_pad, F), bf16),
                   jax.ShapeDtypeStruct((N, F), f32),
                   jax.ShapeDtypeStruct((8, F), f32)),
        grid=(n_tiles,),
        in_specs=[
            pl.BlockSpec((N, F), lambda t: (0, 0)),
            pl.BlockSpec((N, F), lambda t: (0, 0)),
            pl.BlockSpec((TE, F), lambda t: (t, 0)),
            pl.BlockSpec((1, 2, TE // 2), lambda t: (t, 0, 0)),
            pl.BlockSpec((1, 2, TE // 2), lambda t: (t, 0, 0)),
            pl.BlockSpec((F, 2 * F), lambda t: (0, 0)),
            pl.BlockSpec((F, F), lambda t: (0, 0)),
            pl.BlockSpec((F, F), lambda t: (0, 0)),
            pl.BlockSpec((1, F), lambda t: (0, 0)),
            pl.BlockSpec((1, F), lambda t: (0, 0)),
            pl.BlockSpec((F, F), lambda t: (0, 0)),
            pl.BlockSpec((1, F), lambda t: (0, 0)),
            pl.BlockSpec((1, F), lambda t: (0, 0)),
            pl.BlockSpec((1, F), lambda t: (0, 0)),
            pl.BlockSpec((1, F), lambda t: (0, 0)),
            pl.BlockSpec((1, F), lambda t: (0, 0)),
        ],
        out_specs=(
            pl.BlockSpec((TE, F), lambda t: (t, 0)),
            pl.BlockSpec((N, F), lambda t: (0, 0)),
            pl.BlockSpec((8, F), lambda t: (0, 0)),
        ),
        scratch_shapes=[pltpu.VMEM((N, 2 * F), f32),
                        pltpu.VMEM((8, F), f32)],
        compiler_params=pltpu.CompilerParams(
            dimension_semantics=("arbitrary",),
            vmem_limit_bytes=56 * 1024 * 1024),
    )(node_bf, node_feats, edge_p, src_r, dst_r,
      wa, w_dst_gate.astype(bf16), w_edge_gate.astype(bf16), bm, b_dst_update,
      w_src_update.astype(bf16), b_src_update,
      bn_nodes_gamma, bn_nodes_beta, bn_edges_gamma, bn_edges_beta)

    tb = _pick_tile(E_pad, cap=4096)
    if tb == 0:
        tb = E_pad
    y = pl.pallas_call(
        _edge_out_kernel,
        out_shape=jax.ShapeDtypeStruct((E_pad, F), f32),
        grid=(E_pad // tb,),
        in_specs=[
            pl.BlockSpec((tb, F), lambda t: (t, 0)),
            pl.BlockSpec((tb, F), lambda t: (t, 0)),
            pl.BlockSpec((8, F), lambda t: (0, 0)),
        ],
        out_specs=pl.BlockSpec((tb, F), lambda t: (t, 0)),
        compiler_params=pltpu.CompilerParams(
            dimension_semantics=("arbitrary",)),
    )(edge_p, m, ss)
    if padded:
        y = y[:E]
    return x, y


# speedup vs baseline: 1.7095x; 1.1587x over previous
import functools

import jax
import jax.numpy as jnp
from jax import lax
from jax.experimental import pallas as pl
from jax.experimental.pallas import tpu as pltpu

_DIV_EPS = 1e-6
_BN_EPS = 1e-5


def _recip(x):
    r = pl.reciprocal(x, approx=True)
    return r * (2.0 - x * r)


def _pass1_kernel(nodebf_ref, node_ref, edge_ref, src_ref, dstr_ref,
                  wa_ref, wdg_ref, weg_ref, bm_ref, bdu_ref,
                  wsu_ref, bsu_ref,
                  gn_ref, bn_ref, ge_ref, be_ref,
                  m_out, x_out, ss_out,
                  acc_ref, st_ref,
                  *, num_edges, edge_tile, padded):
    f32 = jnp.float32
    bf16 = jnp.bfloat16
    f8 = jnp.float8_e4m3fn
    N = nodebf_ref.shape[0]
    F = wdg_ref.shape[0]
    TE = edge_tile
    t = pl.program_id(0)
    last = pl.num_programs(0) - 1

    @pl.when(t == 0)
    def _init():
        acc_ref[...] = jnp.zeros_like(acc_ref)
        st_ref[...] = jnp.zeros_like(st_ref)

    TE2 = TE // 2
    subl = lax.broadcasted_iota(jnp.int32, (N, TE2), 0)
    dnt = (((0,), (0,)), ((), ()))
    scat = []
    for h in range(2):
        rows = pl.ds(h * TE2, TE2)
        St = (subl == src_ref[0, h:h + 1, :]).astype(bf16)
        eq_d = subl == dstr_ref[0, h:h + 1, :]
        Dt = eq_d.astype(bf16)
        Dt8 = eq_d.astype(f8)

        gS = lax.dot_general(St, nodebf_ref[...], dnt,
                             preferred_element_type=f32)
        gD = lax.dot_general(Dt, nodebf_ref[...], dnt,
                             preferred_element_type=f32)

        P1 = jnp.dot(gS.astype(bf16), wa_ref[...],
                     preferred_element_type=f32)
        P2 = jnp.dot(gD.astype(bf16), wdg_ref[...],
                     preferred_element_type=f32)
        P3 = jnp.dot(edge_ref[rows, :].astype(bf16), weg_ref[...],
                     preferred_element_type=f32)
        m = P1[:, 0:F] + P2 + P3 + bm_ref[...]
        m_out[rows, :] = m.astype(bf16)

        if padded:
            eidx = (t * TE + h * TE2
                    + lax.broadcasted_iota(jnp.int32, (TE2, 1), 0))
            valid = (eidx < num_edges).astype(f32)
            mv = m * valid
        else:
            valid = None
            mv = m

        st_ref[0:1, :] += jnp.sum(mv, axis=0, keepdims=True)
        st_ref[1:2, :] += jnp.sum(mv * m, axis=0, keepdims=True)

        sigma = 0.5 * jnp.tanh(0.5 * m) + 0.5
        if padded:
            sigma = sigma * valid

        Bh = P1[:, F:2 * F] + bdu_ref[...]
        contrib = jnp.concatenate([Bh * sigma, sigma], axis=1).astype(f8)
        scat.append(jnp.dot(Dt8, contrib, preferred_element_type=f32))

    acc_ref[...] += scat[0] + scat[1]

    @pl.when(t == last)
    def _finalize():
        xl = jnp.dot(nodebf_ref[...], wsu_ref[...],
                     preferred_element_type=f32) + bsu_ref[...]
        num = acc_ref[:, 0:F]
        den = acc_ref[:, F:2 * F]
        x = xl + num * _recip(den + _DIV_EPS)
        mean = jnp.mean(x, axis=0, keepdims=True)
        var = jnp.mean((x - mean) ** 2, axis=0, keepdims=True)
        xn = (x - mean) * lax.rsqrt(var + _BN_EPS) * gn_ref[...] + bn_ref[...]
        x_out[...] = node_ref[...] + jnp.clip(xn, 0.0, 6.0)

        inv_e = 1.0 / float(num_edges)
        s = st_ref[0:1, :] * inv_e
        q = st_ref[1:2, :] * inv_e
        var_e = q - s * s
        scale = ge_ref[...] * lax.rsqrt(var_e + _BN_EPS)
        ss_out[0:1, :] = scale
        ss_out[1:2, :] = be_ref[...] - s * scale
        ss_out[2:8, :] = jnp.zeros((6, F), f32)


def _edge_out_kernel(edge_ref, m_ref, ss_ref, y_out):
    y_out[...] = edge_ref[...] + jnp.clip(
        m_ref[...].astype(jnp.float32) * ss_ref[0:1, :] + ss_ref[1:2, :],
        0.0, 6.0)


def _pick_tile(n, cap, step=8):
    best = 0
    d = step
    while d <= min(n, cap):
        if n % d == 0:
            best = d
        d += step
    return best


def kernel(node_feats, edge_feats, src, dst,
           w_src_gate, b_src_gate, w_dst_gate, b_dst_gate,
           w_edge_gate, b_edge_gate, w_src_update, b_src_update,
           w_dst_update, b_dst_update,
           bn_nodes_gamma, bn_nodes_beta, bn_edges_gamma, bn_edges_beta):
    f32 = jnp.float32
    bf16 = jnp.bfloat16
    N, F = node_feats.shape
    E = edge_feats.shape[0]

    TE = _pick_tile(E, cap=2048)
    if TE >= 256:
        E_pad, padded = E, False
        edge_p, src_p, dst_p = edge_feats, src, dst
    else:
        TE = 1024
        E_pad = ((E + TE - 1) // TE) * TE
        padded = True
        edge_p = jnp.pad(edge_feats, ((0, E_pad - E), (0, 0)))
        src_p = jnp.pad(src, (0, E_pad - E))
        dst_p = jnp.pad(dst, (0, E_pad - E))
    n_t = E_pad // TE
    src_r = src_p.astype(jnp.int32).reshape(n_t, 2, TE // 2)
    dst_r = dst_p.astype(jnp.int32).reshape(n_t, 2, TE // 2)
    n_tiles = E_pad // TE

    node_bf = node_feats.astype(bf16)
    wa = jnp.concatenate([w_src_gate, w_dst_update], axis=1).astype(bf16)
    bm = b_src_gate + b_dst_gate + b_edge_gate

    m, x, ss = pl.pallas_call(
        functools.partial(_pass1_kernel, num_edges=E, edge_tile=TE,
                          padded=padded),
        out_shape=(jax.ShapeDtypeStruct((E_pad, F), bf16),
                   jax.ShapeDtypeStruct((N, F), f32),
                   jax.ShapeDtypeStruct((8, F), f32)),
        grid=(n_tiles,),
        in_specs=[
            pl.BlockSpec((N, F), lambda t: (0, 0)),
            pl.BlockSpec((N, F), lambda t: (0, 0)),
            pl.BlockSpec((TE, F), lambda t: (t, 0)),
            pl.BlockSpec((1, 2, TE // 2), lambda t: (t, 0, 0)),
            pl.BlockSpec((1, 2, TE // 2), lambda t: (t, 0, 0)),
            pl.BlockSpec((F, 2 * F), lambda t: (0, 0)),
            pl.BlockSpec((F, F), lambda t: (0, 0)),
            pl.BlockSpec((F, F), lambda t: (0, 0)),
            pl.BlockSpec((1, F), lambda t: (0, 0)),
            pl.BlockSpec((1, F), lambda t: (0, 0)),
            pl.BlockSpec((F, F), lambda t: (0, 0)),
            pl.BlockSpec((1, F), lambda t: (0, 0)),
            pl.BlockSpec((1, F), lambda t: (0, 0)),
            pl.BlockSpec((1, F), lambda t: (0, 0)),
            pl.BlockSpec((1, F), lambda t: (0, 0)),
            pl.BlockSpec((1, F), lambda t: (0, 0)),
        ],
        out_specs=(
            pl.BlockSpec((TE, F), lambda t: (t, 0)),
            pl.BlockSpec((N, F), lambda t: (0, 0)),
            pl.BlockSpec((8, F), lambda t: (0, 0)),
        ),
        scratch_shapes=[pltpu.VMEM((N, 2 * F), f32),
                        pltpu.VMEM((8, F), f32)],
        compiler_params=pltpu.CompilerParams(
            dimension_semantics=("arbitrary",),
            vmem_limit_bytes=56 * 1024 * 1024),
    )(node_bf, node_feats, edge_p, src_r, dst_r,
      wa, w_dst_gate.astype(bf16), w_edge_gate.astype(bf16), bm, b_dst_update,
      w_src_update.astype(bf16), b_src_update,
      bn_nodes_gamma, bn_nodes_beta, bn_edges_gamma, bn_edges_beta)

    tb = _pick_tile(E_pad, cap=4096)
    if tb == 0:
        tb = E_pad
    y = pl.pallas_call(
        _edge_out_kernel,
        out_shape=jax.ShapeDtypeStruct((E_pad, F), f32),
        grid=(E_pad // tb,),
        in_specs=[
            pl.BlockSpec((tb, F), lambda t: (t, 0)),
            pl.BlockSpec((tb, F), lambda t: (t, 0)),
            pl.BlockSpec((8, F), lambda t: (0, 0)),
        ],
        out_specs=pl.BlockSpec((tb, F), lambda t: (t, 0)),
        compiler_params=pltpu.CompilerParams(
            dimension_semantics=("arbitrary",)),
    )(edge_p, m, ss)
    if padded:
        y = y[:E]
    return x, y
